# baseline (speedup 1.0000x reference)
import functools

import jax
import jax.numpy as jnp
from jax import lax
from jax.experimental import pallas as pl
from jax.experimental.pallas import tpu as pltpu
from jax.experimental.pallas import tpu_sc as plsc

NV = 262144
C = 128
NC = 2
NS = 16
NW = NC * NS
PER_W = NV // NW
K = 128
CHUNKS = PER_W // K
NB = 4
W4, W8, W16 = 93, 47, 24
DW = 1220


def _body(t4, t8, t16, dep, px_h, py_h, pz_h, fov_h, free_h, occ_h,
          rows_o, mask_o,
          pxv, pyv, pzv, fovv,
          i4v, i8v, i16v, idv,
          rv, dvv, mbuf, abuf, gbuf, hbuf, freev, occv,
          s_in0, s_in1, s_in2, s_in3,
          sga0, sga1, sga2, sga3,
          sgb0, sgb1, sgb2, sgb3,
          so0, so1, so2, so3):
    sga = [sga0, sga1, sga2, sga3]
    sgb = [sgb0, sgb1, sgb2, sgb3]
    so = [so0, so1, so2, so3]
    wid = lax.axis_index("s") * NC + lax.axis_index("c")
    vbase = wid * PER_W

    c0 = pltpu.async_copy(px_h.at[pl.ds(vbase, PER_W)], pxv, s_in0)
    c1 = pltpu.async_copy(py_h.at[pl.ds(vbase, PER_W)], pyv, s_in1)
    c2 = pltpu.async_copy(pz_h.at[pl.ds(vbase, PER_W)], pzv, s_in2)
    c3 = pltpu.async_copy(fov_h.at[pl.ds(vbase, PER_W)], fovv, s_in3)
    pltpu.sync_copy(free_h, freev)
    pltpu.sync_copy(occ_h, occv)
    c0.wait(); c1.wait(); c2.wait(); c3.wait()
    f_regs = [freev[pl.ds(t * 16, 16)] for t in range(8)]
    o_regs = [occv[pl.ds(t * 16, 16)] for t in range(8)]

    def p1(i, b):
        off = i * K

        @plsc.parallel_loop(0, K // 16, unroll=2)
        def idxbody(j):
            sl = pl.ds(j * 16, 16)
            x = pxv[pl.ds(off + j * 16, 16)]
            y = pyv[pl.ds(off + j * 16, 16)]
            i4v[b, sl] = (y >> 2) * W4 + (x >> 2)
            i8v[b, sl] = (y >> 3) * W8 + (x >> 3)
            i16v[b, sl] = (y >> 4) * W16 + (x >> 4)
            idv[b, sl] = y * DW + x
        pltpu.async_copy(t4.at[i4v.at[b]], rv.at[b], sga[b])

    def p2(i, b):
        pltpu.make_async_copy(t4.at[i4v.at[b]], rv.at[b], sga[b]).wait()
        pltpu.async_copy(t8.at[i8v.at[b]], rv.at[b], sgb[b], add=True)
        pltpu.async_copy(t16.at[i16v.at[b]], rv.at[b], sgb[b], add=True)
        pltpu.async_copy(dep.at[idv.at[b]], dvv.at[b], sgb[b])

    def p3(i, b):
        off = i * K
        pltpu.make_async_copy(t8.at[i8v.at[b]], rv.at[b], sgb[b]).wait()
        pltpu.make_async_copy(t16.at[i16v.at[b]], rv.at[b], sgb[b]).wait()
        pltpu.make_async_copy(dep.at[idv.at[b]], dvv.at[b], sgb[b]).wait()

        @plsc.parallel_loop(0, K // 16, unroll=2)
        def scal(j):
            sl = pl.ds(j * 16, 16)
            d = dvv[b, sl]
            pz = pzv[pl.ds(off + j * 16, 16)]
            fov = fovv[pl.ds(off + j * 16, 16)] > 0
            vz = (pz - d) / (d + 1e-4)
            b1 = fov & (vz >= 0.5) & (vz <= 1.0)
            b2 = fov & (vz > 1.0) & (vz <= 2.0)
            b3 = fov & (vz > 2.0)
            b4 = fov & (vz < 0.5)
            vsafe = jnp.where(b2, vz, 1.0)
            r = 1.0 / vsafe
            abuf[sl] = jnp.where(b1, vz, jnp.where(b2, r, 0.0))
            gbuf[sl] = jnp.where(b4, 1.0, jnp.where(b1, 1.0 - vz, 0.0))
            hbuf[sl] = jnp.where(b3, 1.0, jnp.where(b2, 1.0 - r, 0.0))
            ones = jnp.full((16,), 1, jnp.int32)
            zeros = jnp.full((16,), 0, jnp.int32)
            mbuf[b, sl] = jnp.where(fov & (vz >= 0.4), ones, zeros)

        @plsc.parallel_loop(0, K // 16)
        def blend(j):
            gsl = pl.ds(j * 16, 16)
            a16 = abuf[gsl]
            g16 = gbuf[gsl]
            h16 = hbuf[gsl]
            for u in range(16):
                k = j * 16 + u
                av = a16[u]
                gv = g16[u]
                hv = h16[u]
                for t in range(8):
                    sl = pl.ds(t * 16, 16)
                    rv[b, k, sl] = av * rv[b, k, sl] + gv * f_regs[t] + hv * o_regs[t]

        pltpu.async_copy(rv.at[b], rows_o.at[pl.ds(vbase + off, K)], so[b])
        pltpu.async_copy(mbuf.at[b], mask_o.at[pl.ds(vbase + off, K)], so[b])

    def wait_out(i, b):
        off = i * K
        pltpu.make_async_copy(rv.at[b], rows_o.at[pl.ds(vbase + off, K)], so[b]).wait()
        pltpu.make_async_copy(mbuf.at[b], mask_o.at[pl.ds(vbase + off, K)], so[b]).wait()

    def steady(g, _):
        for bb in range(NB):
            it = g * NB + bb

            @pl.when((it >= 3) & (it <= CHUNKS + 2))
            def _():
                p3(it - 3, (bb + 1) % NB)

            @pl.when((it >= 4) & (it <= CHUNKS + 3))
            def _():
                wait_out(it - 4, bb)

            @pl.when(it <= CHUNKS - 1)
            def _():
                p1(it, bb)

            @pl.when((it >= 1) & (it <= CHUNKS))
            def _():
                p2(it - 1, (bb + 3) % NB)
        return 0
    lax.fori_loop(0, CHUNKS // NB + 1, steady, 0)


_mesh = plsc.VectorSubcoreMesh(core_axis_name="c", subcore_axis_name="s")

_sc_call = functools.partial(
    pl.kernel,
    out_type=[
        jax.ShapeDtypeStruct((NV, C), jnp.float32),
        jax.ShapeDtypeStruct((NV,), jnp.int32),
    ],
    mesh=_mesh,
    scratch_types=[
        pltpu.VMEM((PER_W,), jnp.int32),
        pltpu.VMEM((PER_W,), jnp.int32),
        pltpu.VMEM((PER_W,), jnp.float32),
        pltpu.VMEM((PER_W,), jnp.int32),
        pltpu.VMEM((NB, K), jnp.int32),
        pltpu.VMEM((NB, K), jnp.int32),
        pltpu.VMEM((NB, K), jnp.int32),
        pltpu.VMEM((NB, K), jnp.int32),
        pltpu.VMEM((NB, K, C), jnp.float32),
        pltpu.VMEM((NB, K), jnp.float32),
        pltpu.VMEM((NB, K), jnp.int32),
        pltpu.VMEM((K,), jnp.float32),
        pltpu.VMEM((K,), jnp.float32),
        pltpu.VMEM((K,), jnp.float32),
        pltpu.VMEM((C,), jnp.float32),
        pltpu.VMEM((C,), jnp.float32),
    ] + [pltpu.SemaphoreType.DMA] * 16,
)(_body)


def kernel(feats, depth, projected_pix, pix_z, fov_mask, occ_embed_weight):
    t4 = feats[0, 0][:, :W4, :W4].reshape(C, -1).T
    t8 = feats[1, 0][:, :W8, :W8].reshape(C, -1).T
    t16 = feats[2, 0][:, :W16, :W16].reshape(C, -1).T
    dep = depth.reshape(-1)
    px = projected_pix[0, :, 0].astype(jnp.int32)
    py = projected_pix[0, :, 1].astype(jnp.int32)
    pz = pix_z[0]
    fov = fov_mask[0].astype(jnp.int32)
    free = occ_embed_weight[0]
    occ = occ_embed_weight[1]
    rows, mask = _sc_call(t4, t8, t16, dep, px, py, pz, fov, free, occ)
    out = rows.reshape(128, 128, 16, C).transpose(3, 0, 1, 2)[None]
    return out, (mask != 0).reshape(1, NV)


# speedup vs baseline: 1.2438x; 1.2438x over previous
import functools

import jax
import jax.numpy as jnp
from jax import lax
from jax.experimental import pallas as pl
from jax.experimental.pallas import tpu as pltpu
from jax.experimental.pallas import tpu_sc as plsc

NV = 262144
C = 128
NC = 2
NS = 16
NW = NC * NS
PER_W = NV // NW
K = 64
CHUNKS = PER_W // K
NB = 4
W4, W8, W16 = 93, 47, 24
R4P = 8704
DW = 1220


def _body(t4, t8, t16, dep, px_h, py_h, pz_h, fov_h, free_h, occ_h,
          rows_o, mask_o,
          pxv, pyv, pzv, fovv,
          i4v, i8v, i16v, idv,
          rv, dvv, mbuf, abuf, gbuf, hbuf, freev, occv,
          t4s,
          spp0, spp1, spp2, spp3,
          sga0, sga1, sga2, sga3,
          sgb0, sgb1, sgb2, sgb3,
          so0, so1, so2, so3):
    spp = [spp0, spp1, spp2, spp3]
    sga = [sga0, sga1, sga2, sga3]
    sgb = [sgb0, sgb1, sgb2, sgb3]
    so = [so0, so1, so2, so3]
    wid = lax.axis_index("s") * NC + lax.axis_index("c")
    vbase = wid * PER_W

    pltpu.sync_copy(free_h, freev)
    pltpu.sync_copy(occ_h, occv)
    f_regs = [freev[pl.ds(t * 16, 16)] for t in range(8)]
    o_regs = [occv[pl.ds(t * 16, 16)] for t in range(8)]

    sid = lax.axis_index("s")
    for s in range(NS):
        @pl.when(sid == s)
        def _():
            pltpu.sync_copy(t4.at[pl.ds(s * (R4P // NS), R4P // NS)],
                            t4s.at[pl.ds(s * (R4P // NS), R4P // NS)])
    plsc.subcore_barrier()

    def pp_issue(i, b):
        off = vbase + i * K
        pltpu.async_copy(px_h.at[pl.ds(off, K)], pxv.at[b], spp[b])
        pltpu.async_copy(py_h.at[pl.ds(off, K)], pyv.at[b], spp[b])

    def pp_wait(i, b):
        off = vbase + i * K
        pltpu.make_async_copy(px_h.at[pl.ds(off, K)], pxv.at[b], spp[b]).wait()
        pltpu.make_async_copy(py_h.at[pl.ds(off, K)], pyv.at[b], spp[b]).wait()

    def p1(i, b):
        pp_wait(i, b)

        def idxbody(j, _):
            sl = pl.ds(j * 16, 16)
            x = pxv[b, sl]
            y = pyv[b, sl]
            i4v[b, sl] = (y >> 2) * W4 + (x >> 2)
            i8v[b, sl] = (y >> 3) * W8 + (x >> 3)
            i16v[b, sl] = (y >> 4) * W16 + (x >> 4)
            idv[b, sl] = y * DW + x
            return 0
        lax.fori_loop(0, K // 16, idxbody, 0)
        pltpu.async_copy(t4s.at[i4v.at[b]], rv.at[b], sga[b])

        @pl.when(i + 1 <= CHUNKS - 1)
        def _():
            pp_issue(i + 1, (b + 1) % NB)

    def p2(i, b):
        off = vbase + i * K
        pltpu.make_async_copy(t4s.at[i4v.at[b]], rv.at[b], sga[b]).wait()
        pltpu.async_copy(t8.at[i8v.at[b]], rv.at[b], sgb[b], add=True)
        pltpu.async_copy(t16.at[i16v.at[b]], rv.at[b], sgb[b], add=True)
        pltpu.async_copy(dep.at[idv.at[b]], dvv.at[b], sgb[b])
        pltpu.async_copy(pz_h.at[pl.ds(off, K)], pzv.at[b], sgb[b])
        pltpu.async_copy(fov_h.at[pl.ds(off, K)], fovv.at[b], sgb[b])

    def p3(i, b):
        off = vbase + i * K
        pltpu.make_async_copy(t8.at[i8v.at[b]], rv.at[b], sgb[b]).wait()
        pltpu.make_async_copy(t16.at[i16v.at[b]], rv.at[b], sgb[b]).wait()
        pltpu.make_async_copy(dep.at[idv.at[b]], dvv.at[b], sgb[b]).wait()
        pltpu.make_async_copy(pz_h.at[pl.ds(off, K)], pzv.at[b], sgb[b]).wait()
        pltpu.make_async_copy(fov_h.at[pl.ds(off, K)], fovv.at[b], sgb[b]).wait()

        def scal(j, _):
            sl = pl.ds(j * 16, 16)
            d = dvv[b, sl]
            pz = pzv[b, sl]
            fov = fovv[b, sl] > 0
            vz = (pz - d) / (d + 1e-4)
            b1 = fov & (vz >= 0.5) & (vz <= 1.0)
            b2 = fov & (vz > 1.0) & (vz <= 2.0)
            b3 = fov & (vz > 2.0)
            b4 = fov & (vz < 0.5)
            vsafe = jnp.where(b2, vz, 1.0)
            r = 1.0 / vsafe
            abuf[sl] = jnp.where(b1, vz, jnp.where(b2, r, 0.0))
            gbuf[sl] = jnp.where(b4, 1.0, jnp.where(b1, 1.0 - vz, 0.0))
            hbuf[sl] = jnp.where(b3, 1.0, jnp.where(b2, 1.0 - r, 0.0))
            ones = jnp.full((16,), 1, jnp.int32)
            zeros = jnp.full((16,), 0, jnp.int32)
            mbuf[b, sl] = jnp.where(fov & (vz >= 0.4), ones, zeros)
            return 0
        lax.fori_loop(0, K // 16, scal, 0)

        def blend(j, _):
            gsl = pl.ds(j * 16, 16)
            a16 = abuf[gsl]
            g16 = gbuf[gsl]
            h16 = hbuf[gsl]
            for u in range(16):
                k = j * 16 + u
                av = a16[u]
                gv = g16[u]
                hv = h16[u]
                for t in range(8):
                    sl = pl.ds(t * 16, 16)
                    rv[b, k, sl] = av * rv[b, k, sl] + gv * f_regs[t] + hv * o_regs[t]
            return 0
        lax.fori_loop(0, K // 16, blend, 0)

        pltpu.async_copy(rv.at[b], rows_o.at[pl.ds(off, K)], so[b])
        pltpu.async_copy(mbuf.at[b], mask_o.at[pl.ds(off, K)], so[b])

    def wait_out(i, b):
        off = vbase + i * K
        pltpu.make_async_copy(rv.at[b], rows_o.at[pl.ds(off, K)], so[b]).wait()
        pltpu.make_async_copy(mbuf.at[b], mask_o.at[pl.ds(off, K)], so[b]).wait()

    pp_issue(0, 0)

    def steady(g, _):
        for bb in range(NB):
            it = g * NB + bb

            @pl.when((it >= 3) & (it <= CHUNKS + 2))
            def _():
                p3(it - 3, (bb + 1) % NB)

            @pl.when((it >= 4) & (it <= CHUNKS + 3))
            def _():
                wait_out(it - 4, bb)

            @pl.when(it <= CHUNKS - 1)
            def _():
                p1(it, bb)

            @pl.when((it >= 1) & (it <= CHUNKS))
            def _():
                p2(it - 1, (bb + 3) % NB)
        return 0
    lax.fori_loop(0, (CHUNKS + 3) // NB + 1, steady, 0)


_mesh = plsc.VectorSubcoreMesh(core_axis_name="c", subcore_axis_name="s")

_sc_call = functools.partial(
    pl.kernel,
    out_type=[
        jax.ShapeDtypeStruct((NV, C), jnp.float32),
        jax.ShapeDtypeStruct((NV,), jnp.int32),
    ],
    mesh=_mesh,
    scratch_types=[
        pltpu.VMEM((NB, K), jnp.int32),
        pltpu.VMEM((NB, K), jnp.int32),
        pltpu.VMEM((NB, K), jnp.float32),
        pltpu.VMEM((NB, K), jnp.int32),
        pltpu.VMEM((NB, K), jnp.int32),
        pltpu.VMEM((NB, K), jnp.int32),
        pltpu.VMEM((NB, K), jnp.int32),
        pltpu.VMEM((NB, K), jnp.int32),
        pltpu.VMEM((NB, K, C), jnp.float32),
        pltpu.VMEM((NB, K), jnp.float32),
        pltpu.VMEM((NB, K), jnp.int32),
        pltpu.VMEM((K,), jnp.float32),
        pltpu.VMEM((K,), jnp.float32),
        pltpu.VMEM((K,), jnp.float32),
        pltpu.VMEM((C,), jnp.float32),
        pltpu.VMEM((C,), jnp.float32),
        pltpu.VMEM_SHARED((R4P, C), jnp.float32),
    ] + [pltpu.SemaphoreType.DMA] * 16,
)(_body)


def kernel(feats, depth, projected_pix, pix_z, fov_mask, occ_embed_weight):
    t4 = feats[0, 0][:, :W4, :W4].reshape(C, -1).T
    t8 = feats[1, 0][:, :W8, :W8].reshape(C, -1).T
    t16 = feats[2, 0][:, :W16, :W16].reshape(C, -1).T
    t4 = jnp.pad(t4, ((0, R4P - t4.shape[0]), (0, 0)))
    dep = depth.reshape(-1)
    px = projected_pix[0, :, 0].astype(jnp.int32)
    py = projected_pix[0, :, 1].astype(jnp.int32)
    pz = pix_z[0]
    fov = fov_mask[0].astype(jnp.int32)
    free = occ_embed_weight[0]
    occ = occ_embed_weight[1]
    rows, mask = _sc_call(t4, t8, t16, dep, px, py, pz, fov, free, occ)
    out = rows.reshape(128, 128, 16, C).transpose(3, 0, 1, 2)[None]
    return out, (mask != 0).reshape(1, NV)


# speedup vs baseline: 1.2532x; 1.0075x over previous
import functools

import jax
import jax.numpy as jnp
from jax import lax
from jax.experimental import pallas as pl
from jax.experimental.pallas import tpu as pltpu
from jax.experimental.pallas import tpu_sc as plsc

NV = 262144
C = 128
NC = 2
NS = 16
NW = NC * NS
PER_W = NV // NW
K = 64
CHUNKS = PER_W // K
NB = 4
W4, W8, W16 = 93, 47, 24
R4P = 8704
DW = 1220


def _body(t4, t8, t16, dep, px_h, py_h, pz_h, fov_h, free_h, occ_h,
          rows_o, mask_o,
          pxv, pyv, pzv, fovv,
          i4v, i8v, i16v, idv,
          rv, dvv, mbuf, abuf, gbuf, hbuf, freev, occv,
          t4s,
          spp0, spp1, spp2, spp3,
          sga0, sga1, sga2, sga3,
          sgb0, sgb1, sgb2, sgb3,
          so0, so1, so2, so3):
    spp = [spp0, spp1, spp2, spp3]
    sga = [sga0, sga1, sga2, sga3]
    sgb = [sgb0, sgb1, sgb2, sgb3]
    so = [so0, so1, so2, so3]
    wid = lax.axis_index("s") * NC + lax.axis_index("c")
    vbase = wid * PER_W

    pltpu.sync_copy(free_h, freev)
    pltpu.sync_copy(occ_h, occv)
    f_regs = [freev[pl.ds(t * 16, 16)] for t in range(8)]
    o_regs = [occv[pl.ds(t * 16, 16)] for t in range(8)]

    sid = lax.axis_index("s")
    for s in range(NS):
        @pl.when(sid == s)
        def _():
            pltpu.sync_copy(t4.at[pl.ds(s * (R4P // NS), R4P // NS)],
                            t4s.at[pl.ds(s * (R4P // NS), R4P // NS)])
    plsc.subcore_barrier()

    def pp_issue(i, b):
        off = vbase + i * K
        pltpu.async_copy(px_h.at[pl.ds(off, K)], pxv.at[b], spp[b])
        pltpu.async_copy(py_h.at[pl.ds(off, K)], pyv.at[b], spp[b])

    def pp_wait(i, b):
        off = vbase + i * K
        pltpu.make_async_copy(px_h.at[pl.ds(off, K)], pxv.at[b], spp[b]).wait()
        pltpu.make_async_copy(py_h.at[pl.ds(off, K)], pyv.at[b], spp[b]).wait()

    def p1(i, b):
        pp_wait(i, b)

        def idxbody(j, _):
            sl = pl.ds(j * 16, 16)
            x = pxv[b, sl]
            y = pyv[b, sl]
            i4v[b, sl] = (y >> 2) * W4 + (x >> 2)
            i8v[b, sl] = (y >> 3) * W8 + (x >> 3)
            i16v[b, sl] = (y >> 4) * W16 + (x >> 4)
            idv[b, sl] = y * DW + x
            return 0
        lax.fori_loop(0, K // 16, idxbody, 0)
        pltpu.async_copy(t4s.at[i4v.at[b]], rv.at[b], sga[b])

        @pl.when(i + 1 <= CHUNKS - 1)
        def _():
            pp_issue(i + 1, (b + 1) % NB)

    def p2(i, b):
        off = vbase + i * K
        pltpu.make_async_copy(t4s.at[i4v.at[b]], rv.at[b], sga[b]).wait()
        pltpu.async_copy(t8.at[i8v.at[b]], rv.at[b], sgb[b], add=True)
        pltpu.async_copy(t16.at[i16v.at[b]], rv.at[b], sgb[b], add=True)
        pltpu.async_copy(dep.at[idv.at[b]], dvv.at[b], sgb[b])
        pltpu.async_copy(pz_h.at[pl.ds(off, K)], pzv.at[b], sgb[b])
        pltpu.async_copy(fov_h.at[pl.ds(off, K)], fovv.at[b], sgb[b])

    def p3(i, b):
        off = vbase + i * K
        pltpu.make_async_copy(t8.at[i8v.at[b]], rv.at[b], sgb[b]).wait()
        pltpu.make_async_copy(t16.at[i16v.at[b]], rv.at[b], sgb[b]).wait()
        pltpu.make_async_copy(dep.at[idv.at[b]], dvv.at[b], sgb[b]).wait()
        pltpu.make_async_copy(pz_h.at[pl.ds(off, K)], pzv.at[b], sgb[b]).wait()
        pltpu.make_async_copy(fov_h.at[pl.ds(off, K)], fovv.at[b], sgb[b]).wait()

        def scal(j, _):
            sl = pl.ds(j * 16, 16)
            d = dvv[b, sl]
            pz = pzv[b, sl]
            fov = fovv[b, sl] > 0
            vz = (pz - d) / (d + 1e-4)
            b1 = fov & (vz >= 0.5) & (vz <= 1.0)
            b2 = fov & (vz > 1.0) & (vz <= 2.0)
            b3 = fov & (vz > 2.0)
            b4 = fov & (vz < 0.5)
            vsafe = jnp.where(b2, vz, 1.0)
            r = 1.0 / vsafe
            abuf[sl] = jnp.where(b1, vz, jnp.where(b2, r, 0.0))
            gg = jnp.where(b4, 1.0, jnp.where(b1, 1.0 - vz, 0.0))
            hh = jnp.where(b3, 1.0, jnp.where(b2, 1.0 - r, 0.0))
            gbuf[sl] = gg + hh
            hbuf[sl] = jnp.where(b2 | b3, 1.0, 0.0)
            ones = jnp.full((16,), 1, jnp.int32)
            zeros = jnp.full((16,), 0, jnp.int32)
            mbuf[b, sl] = jnp.where(fov & (vz >= 0.4), ones, zeros)
            return 0
        lax.fori_loop(0, K // 16, scal, 0)

        def blend(j, _):
            gsl = pl.ds(j * 16, 16)
            a16 = abuf[gsl]
            g16 = gbuf[gsl]
            h16 = hbuf[gsl]
            for u in range(16):
                k = j * 16 + u
                av = a16[u]
                sv = g16[u]
                occ_sel = h16[u] > 0.0
                for t in range(8):
                    sl = pl.ds(t * 16, 16)
                    e = jnp.where(occ_sel, o_regs[t], f_regs[t])
                    rv[b, k, sl] = av * rv[b, k, sl] + sv * e
            return 0
        lax.fori_loop(0, K // 16, blend, 0)

        pltpu.async_copy(rv.at[b], rows_o.at[pl.ds(off, K)], so[b])
        pltpu.async_copy(mbuf.at[b], mask_o.at[pl.ds(off, K)], so[b])

    def wait_out(i, b):
        off = vbase + i * K
        pltpu.make_async_copy(rv.at[b], rows_o.at[pl.ds(off, K)], so[b]).wait()
        pltpu.make_async_copy(mbuf.at[b], mask_o.at[pl.ds(off, K)], so[b]).wait()

    pp_issue(0, 0)

    def steady(g, _):
        for bb in range(NB):
            it = g * NB + bb

            @pl.when((it >= 3) & (it <= CHUNKS + 2))
            def _():
                p3(it - 3, (bb + 1) % NB)

            @pl.when((it >= 4) & (it <= CHUNKS + 3))
            def _():
                wait_out(it - 4, bb)

            @pl.when(it <= CHUNKS - 1)
            def _():
                p1(it, bb)

            @pl.when((it >= 1) & (it <= CHUNKS))
            def _():
                p2(it - 1, (bb + 3) % NB)
        return 0
    lax.fori_loop(0, (CHUNKS + 3) // NB + 1, steady, 0)


_mesh = plsc.VectorSubcoreMesh(core_axis_name="c", subcore_axis_name="s")

_sc_call = functools.partial(
    pl.kernel,
    out_type=[
        jax.ShapeDtypeStruct((NV, C), jnp.float32),
        jax.ShapeDtypeStruct((NV,), jnp.int32),
    ],
    mesh=_mesh,
    scratch_types=[
        pltpu.VMEM((NB, K), jnp.int32),
        pltpu.VMEM((NB, K), jnp.int32),
        pltpu.VMEM((NB, K), jnp.float32),
        pltpu.VMEM((NB, K), jnp.int32),
        pltpu.VMEM((NB, K), jnp.int32),
        pltpu.VMEM((NB, K), jnp.int32),
        pltpu.VMEM((NB, K), jnp.int32),
        pltpu.VMEM((NB, K), jnp.int32),
        pltpu.VMEM((NB, K, C), jnp.float32),
        pltpu.VMEM((NB, K), jnp.float32),
        pltpu.VMEM((NB, K), jnp.int32),
        pltpu.VMEM((K,), jnp.float32),
        pltpu.VMEM((K,), jnp.float32),
        pltpu.VMEM((K,), jnp.float32),
        pltpu.VMEM((C,), jnp.float32),
        pltpu.VMEM((C,), jnp.float32),
        pltpu.VMEM_SHARED((R4P, C), jnp.float32),
    ] + [pltpu.SemaphoreType.DMA] * 16,
)(_body)


def kernel(feats, depth, projected_pix, pix_z, fov_mask, occ_embed_weight):
    t4 = feats[0, 0][:, :W4, :W4].reshape(C, -1).T
    t8 = feats[1, 0][:, :W8, :W8].reshape(C, -1).T
    t16 = feats[2, 0][:, :W16, :W16].reshape(C, -1).T
    t4 = jnp.pad(t4, ((0, R4P - t4.shape[0]), (0, 0)))
    dep = depth.reshape(-1)
    px = projected_pix[0, :, 0].astype(jnp.int32)
    py = projected_pix[0, :, 1].astype(jnp.int32)
    pz = pix_z[0]
    fov = fov_mask[0].astype(jnp.int32)
    free = occ_embed_weight[0]
    occ = occ_embed_weight[1]
    rows, mask = _sc_call(t4, t8, t16, dep, px, py, pz, fov, free, occ)
    out = rows.reshape(128, 128, 16, C).transpose(3, 0, 1, 2)[None]
    return out, (mask != 0).reshape(1, NV)
